# baseline (speedup 1.0000x reference)
import jax
import jax.numpy as jnp
from jax import lax
from jax.experimental import pallas as pl
from jax.experimental.pallas import tpu as pltpu
from jax.experimental.pallas import tpu_sc as plsc

TOT = 128
D = 4096
BATCH = 128
B = BATCH * TOT

_info = plsc.get_sparse_core_info()
NC, NS = _info.num_cores, _info.num_subcores
NW = NC * NS
B_PER_W = B // NW
S_ROWS = 128
SE = S_ROWS // 8
NDB = (B_PER_W - S_ROWS) // 16
STEPS = SE // 2


def _body(idx_hbm, table_hbm, out_hbm, idx_v, table_sp, buf0, buf1,
          dsem, g0, g1, s0, s1):
    sid = lax.axis_index("s")
    wid = sid * NC + lax.axis_index("c")
    base = wid * B_PER_W
    pltpu.sync_copy(idx_hbm.at[wid], idx_v)
    rpw = TOT // NS
    pltpu.sync_copy(table_hbm.at[pl.ds(sid * rpw, rpw)],
                    table_sp.at[pl.ds(sid * rpw, rpw)])
    plsc.subcore_barrier()

    def fire_direct(g):
        vec = idx_v[pl.ds(S_ROWS + g * 16, 16)]
        for jj in range(16):
            pltpu.async_copy(
                table_sp.at[pl.ds(vec[jj], 1)],
                out_hbm.at[pl.ds(base + S_ROWS + g * 16 + jj, 1)], dsem)

    def drain_direct16():
        for jj in range(16):
            pltpu.make_async_copy(table_sp.at[pl.ds(0, 1)],
                                  out_hbm.at[pl.ds(base, 1)], dsem).wait()

    def fire_stage(e, buf, sem):
        vec = idx_v[pl.ds(e * 8, 16)]
        for jj in range(8):
            pltpu.async_copy(table_sp.at[pl.ds(vec[jj], 1)],
                             buf.at[pl.ds(jj, 1)], sem)

    def drain_stage(buf, sem):
        for jj in range(8):
            pltpu.make_async_copy(table_sp.at[pl.ds(0, 1)],
                                  buf.at[pl.ds(jj, 1)], sem).wait()

    def fire_write(e, buf, sem):
        pltpu.async_copy(buf, out_hbm.at[pl.ds(base + e * 8, 8)], sem)

    def wait_write(e, buf, sem):
        pltpu.make_async_copy(buf, out_hbm.at[pl.ds(base + e * 8, 8)],
                              sem).wait()

    fire_stage(0, buf0, g0)
    fire_direct(0)

    def step(i, carry):
        a = 2 * i
        b = a + 1

        @pl.when(i >= 1)
        def _():
            wait_write(b - 2, buf1, s1)

        fire_stage(b, buf1, g1)
        drain_direct16()
        drain_stage(buf0, g0)
        fire_write(a, buf0, s0)
        fire_direct(3 * i + 1)

        wait_write(a, buf0, s0)

        @pl.when(b + 1 < SE)
        def _():
            fire_stage(b + 1, buf0, g0)

        drain_stage(buf1, g1)
        fire_write(b, buf1, s1)
        drain_direct16()
        fire_direct(3 * i + 2)

        @pl.when(3 * i + 3 < NDB)
        def _():
            fire_direct(3 * i + 3)

        drain_direct16()
        return carry

    lax.fori_loop(0, STEPS, step, 0)
    wait_write(SE - 1, buf1, s1)


_gather = pl.kernel(
    _body,
    out_type=jax.ShapeDtypeStruct((B, D), jnp.float32),
    mesh=plsc.VectorSubcoreMesh(core_axis_name="c", subcore_axis_name="s"),
    scratch_types=[
        pltpu.VMEM((B_PER_W,), jnp.int32),
        pltpu.VMEM_SHARED((TOT, D), jnp.float32),
        pltpu.VMEM((8, D), jnp.float32),
        pltpu.VMEM((8, D), jnp.float32),
        pltpu.SemaphoreType.DMA,
        pltpu.SemaphoreType.DMA,
        pltpu.SemaphoreType.DMA,
        pltpu.SemaphoreType.DMA,
        pltpu.SemaphoreType.DMA,
    ],
)


def kernel(indices, embedding):
    idx = indices.astype(jnp.int32).reshape(NW, B_PER_W)
    out = _gather(idx, embedding)
    return out.reshape(BATCH, TOT, D)


# speedup vs baseline: 1.0581x; 1.0581x over previous
import jax
import jax.numpy as jnp
from jax import lax
from jax.experimental import pallas as pl
from jax.experimental.pallas import tpu as pltpu
from jax.experimental.pallas import tpu_sc as plsc

TOT = 128
D = 4096
BATCH = 128
B = BATCH * TOT

_info = plsc.get_sparse_core_info()
NC, NS = _info.num_cores, _info.num_subcores
NW = NC * NS
B_PER_W = B // NW
S_ROWS = 192
SE = S_ROWS // 8
NDB = (B_PER_W - S_ROWS) // 16
STEPS = SE // 2


def _body(idx_hbm, table_hbm, out_hbm, idx_v, table_sp, buf0, buf1,
          dsem, g0, g1, s0, s1):
    sid = lax.axis_index("s")
    wid = sid * NC + lax.axis_index("c")
    base = wid * B_PER_W
    pltpu.sync_copy(idx_hbm.at[wid], idx_v)
    rpw = TOT // NS
    pltpu.sync_copy(table_hbm.at[pl.ds(sid * rpw, rpw)],
                    table_sp.at[pl.ds(sid * rpw, rpw)])
    plsc.subcore_barrier()

    def fire_direct(g):
        vec = idx_v[pl.ds(S_ROWS + g * 16, 16)]
        for jj in range(16):
            pltpu.async_copy(
                table_sp.at[pl.ds(vec[jj], 1)],
                out_hbm.at[pl.ds(base + S_ROWS + g * 16 + jj, 1)], dsem)

    def drain_direct16():
        for jj in range(16):
            pltpu.make_async_copy(table_sp.at[pl.ds(0, 1)],
                                  out_hbm.at[pl.ds(base, 1)], dsem).wait()

    def fire_stage(e, buf, sem):
        vec = idx_v[pl.ds(e * 8, 16)]
        for jj in range(8):
            pltpu.async_copy(table_sp.at[pl.ds(vec[jj], 1)],
                             buf.at[pl.ds(jj, 1)], sem)

    def drain_stage(buf, sem):
        for jj in range(8):
            pltpu.make_async_copy(table_sp.at[pl.ds(0, 1)],
                                  buf.at[pl.ds(jj, 1)], sem).wait()

    def fire_write(e, buf, sem):
        pltpu.async_copy(buf, out_hbm.at[pl.ds(base + e * 8, 8)], sem)

    def wait_write(e, buf, sem):
        pltpu.make_async_copy(buf, out_hbm.at[pl.ds(base + e * 8, 8)],
                              sem).wait()

    fire_stage(0, buf0, g0)
    fire_direct(0)

    def step(i, carry):
        a = 2 * i
        b = a + 1

        @pl.when(i >= 1)
        def _():
            wait_write(b - 2, buf1, s1)

        fire_stage(b, buf1, g1)

        @pl.when(i <= 10)
        def _():
            drain_direct16()

        drain_stage(buf0, g0)
        fire_write(a, buf0, s0)

        @pl.when(i <= 9)
        def _():
            fire_direct(2 * i + 1)

        wait_write(a, buf0, s0)

        @pl.when(b + 1 < SE)
        def _():
            fire_stage(b + 1, buf0, g0)

        drain_stage(buf1, g1)
        fire_write(b, buf1, s1)

        @pl.when(jnp.logical_and(i >= 1, i <= 9))
        def _():
            drain_direct16()

        @pl.when(i <= 8)
        def _():
            fire_direct(2 * i + 2)

        return carry

    lax.fori_loop(0, STEPS, step, 0)
    wait_write(SE - 1, buf1, s1)


_gather = pl.kernel(
    _body,
    out_type=jax.ShapeDtypeStruct((B, D), jnp.float32),
    mesh=plsc.VectorSubcoreMesh(core_axis_name="c", subcore_axis_name="s"),
    scratch_types=[
        pltpu.VMEM((B_PER_W,), jnp.int32),
        pltpu.VMEM_SHARED((TOT, D), jnp.float32),
        pltpu.VMEM((8, D), jnp.float32),
        pltpu.VMEM((8, D), jnp.float32),
        pltpu.SemaphoreType.DMA,
        pltpu.SemaphoreType.DMA,
        pltpu.SemaphoreType.DMA,
        pltpu.SemaphoreType.DMA,
        pltpu.SemaphoreType.DMA,
    ],
)


def kernel(indices, embedding):
    idx = indices.astype(jnp.int32).reshape(NW, B_PER_W)
    out = _gather(idx, embedding)
    return out.reshape(BATCH, TOT, D)
